# Initial kernel scaffold; baseline (speedup 1.0000x reference)
import functools

import jax
import jax.numpy as jnp
from jax import lax
from jax.experimental import pallas as pl
from jax.experimental.pallas import tpu as pltpu
from jax.experimental.pallas import tpu_sc as plsc

N = 10000
F = 128
H2 = 256

NC = 2
NS = 16
NW = NC * NS

GW = 128
G = 80
E_PAD = NW * G * GW
N_PAD = 10240
RT = N_PAD // NS


def _make_sc_agg(d, with_counts):
  mesh = plsc.VectorSubcoreMesh(core_axis_name="c", subcore_axis_name="s",
                                num_cores=NC, num_subcores=NS)
  out_type = [jax.ShapeDtypeStruct((NC, N_PAD, d), jnp.float32)]
  scratch = [
      pltpu.VMEM((G, GW), jnp.int32),
      pltpu.VMEM((G, GW), jnp.int32),
      pltpu.VMEM((GW, d), jnp.float32),
      pltpu.VMEM_SHARED((N_PAD, d), jnp.float32),
      pltpu.SemaphoreType.DMA,
  ]
  if with_counts:
    out_type.append(jax.ShapeDtypeStruct((NC, N_PAD, 16), jnp.float32))
    scratch += [
        pltpu.VMEM((GW, 16), jnp.float32),
        pltpu.VMEM_SHARED((N_PAD, 16), jnp.float32),
    ]

  def body(x_hbm, src_hbm, dst_hbm, zacc_hbm, *rest):
    if with_counts:
      (zcnt_hbm, ones_hbm, part_hbm, cnt_hbm,
       src_v, dst_v, rows_v, acc_sh, sem, ones_v, cnt_sh) = rest
    else:
      part_hbm, src_v, dst_v, rows_v, acc_sh, sem = rest
    c = lax.axis_index("c")
    s = lax.axis_index("s")
    wid = c * NS + s

    pltpu.sync_copy(zacc_hbm, acc_sh.at[pl.ds(s * RT, RT)])
    if with_counts:
      pltpu.sync_copy(zcnt_hbm, cnt_sh.at[pl.ds(s * RT, RT)])
      pltpu.sync_copy(ones_hbm, ones_v)
    pltpu.sync_copy(src_hbm.at[pl.ds(wid * G, G)], src_v)
    pltpu.sync_copy(dst_hbm.at[pl.ds(wid * G, G)], dst_v)
    plsc.subcore_barrier()

    @pl.loop(0, G)
    def _(g):
      pltpu.async_copy(x_hbm.at[src_v.at[g]], rows_v, sem).wait()
      pltpu.sync_copy(rows_v, acc_sh.at[dst_v.at[g]], add=True)
      if with_counts:
        pltpu.sync_copy(ones_v, cnt_sh.at[dst_v.at[g]], add=True)

    plsc.subcore_barrier()
    pltpu.sync_copy(acc_sh.at[pl.ds(s * RT, RT)],
                    part_hbm.at[c].at[pl.ds(s * RT, RT)])
    if with_counts:
      pltpu.sync_copy(cnt_sh.at[pl.ds(s * RT, RT)],
                      cnt_hbm.at[c].at[pl.ds(s * RT, RT)])

  return pl.kernel(body, out_type=out_type, mesh=mesh, scratch_types=scratch)


_sc_agg_cnt_128 = _make_sc_agg(F, True)
_sc_agg_128 = _make_sc_agg(F, False)
_sc_agg_16 = _make_sc_agg(16, False)


BR = 400


def _mean(part_ref, cnt_ref):
  agg = part_ref[0] + part_ref[1]
  deg = (cnt_ref[0] + cnt_ref[1])[:, 0:1]
  return agg * (1.0 / jnp.maximum(deg, 1.0))


def _tc_layer1(part_ref, cnt_ref, x_ref, wl_ref, wr_ref, b_ref, o_ref):
  mean = _mean(part_ref, cnt_ref)
  h = jnp.dot(mean, wl_ref[...], preferred_element_type=jnp.float32)
  h = h + b_ref[...] + jnp.dot(x_ref[...], wr_ref[...],
                               preferred_element_type=jnp.float32)
  o_ref[...] = jnp.maximum(h, 0.0)


def _tc_layer2(part_ref, cnt_ref, h1_ref, wl_ref, wr_ref, b_ref, w3l_ref,
               h2_ref, z_ref):
  mean = _mean(part_ref, cnt_ref)
  h = jnp.dot(mean, wl_ref[...], preferred_element_type=jnp.float32)
  h = h + b_ref[...] + jnp.dot(h1_ref[...], wr_ref[...],
                               preferred_element_type=jnp.float32)
  h2 = jnp.maximum(h, 0.0)
  h2_ref[...] = h2
  z_ref[...] = jnp.dot(h2, w3l_ref[...], preferred_element_type=jnp.float32)


def _tc_layer3(part_ref, cnt_ref, h2_ref, w3r_ref, b3_ref, o_ref):
  mean = _mean(part_ref, cnt_ref)
  r = jnp.dot(h2_ref[...], w3r_ref[...], preferred_element_type=jnp.float32)
  logits = (mean + r + b3_ref[...])[:, 0:2]
  m = jnp.max(logits, axis=1, keepdims=True)
  lse = m + jnp.log(jnp.sum(jnp.exp(logits - m), axis=1, keepdims=True))
  o_ref[...] = logits - lse


def _part_spec(d):
  return pl.BlockSpec((NC, BR, d), lambda i: (0, i, 0))


def _full(shape):
  return pl.BlockSpec(shape, lambda i: tuple(0 for _ in shape))


_GRID = (N // BR,)


def kernel(x, edge_index, W1_l, W1_r, b1, W2_l, W2_r, b2, W3_l, W3_r, b3):
  src = edge_index[0]
  dst = edge_index[1]
  npad = E_PAD - src.shape[0]
  src_p = jnp.concatenate([src, jnp.zeros((npad,), jnp.int32)])
  dst_p = jnp.concatenate(
      [dst, N + (jnp.arange(npad, dtype=jnp.int32) % (N_PAD - N))])
  src2d = src_p.reshape(NW * G, GW)
  dst2d = dst_p.reshape(NW * G, GW)

  zacc128 = jnp.zeros((RT, F), jnp.float32)
  zacc16 = jnp.zeros((RT, 16), jnp.float32)
  ones = jnp.ones((GW, 16), jnp.float32)

  part1, cnt = _sc_agg_cnt_128(x, src2d, dst2d, zacc128, zacc16, ones)

  b1r = b1.reshape(1, F)
  h1 = pl.pallas_call(
      _tc_layer1,
      grid=_GRID,
      in_specs=[_part_spec(F), _part_spec(16), pl.BlockSpec((BR, F), lambda i: (i, 0)),
                _full((F, F)), _full((F, F)), _full((1, F))],
      out_specs=pl.BlockSpec((BR, F), lambda i: (i, 0)),
      out_shape=jax.ShapeDtypeStruct((N, F), jnp.float32),
  )(part1, cnt, x, W1_l, W1_r, b1r)

  (part2,) = _sc_agg_128(h1, src2d, dst2d, zacc128)

  b2r = b2.reshape(1, H2)
  w3l_pad = jnp.pad(W3_l, ((0, 0), (0, 14)))
  h2, z = pl.pallas_call(
      _tc_layer2,
      grid=_GRID,
      in_specs=[_part_spec(F), _part_spec(16), pl.BlockSpec((BR, F), lambda i: (i, 0)),
                _full((F, H2)), _full((F, H2)), _full((1, H2)), _full((H2, 16))],
      out_specs=[pl.BlockSpec((BR, H2), lambda i: (i, 0)),
                 pl.BlockSpec((BR, 16), lambda i: (i, 0))],
      out_shape=[jax.ShapeDtypeStruct((N, H2), jnp.float32),
                 jax.ShapeDtypeStruct((N, 16), jnp.float32)],
  )(part2, cnt, h1, W2_l, W2_r, b2r, w3l_pad)

  (part3,) = _sc_agg_16(z, src2d, dst2d, zacc16)

  w3r_pad = jnp.pad(W3_r, ((0, 0), (0, 14)))
  b3_pad = jnp.pad(b3, (0, 14)).reshape(1, 16)
  out = pl.pallas_call(
      _tc_layer3,
      grid=_GRID,
      in_specs=[_part_spec(16), _part_spec(16), pl.BlockSpec((BR, H2), lambda i: (i, 0)),
                _full((H2, 16)), _full((1, 16))],
      out_specs=pl.BlockSpec((BR, 2), lambda i: (i, 0)),
      out_shape=jax.ShapeDtypeStruct((N, 2), jnp.float32),
  )(part3, cnt, h2, w3r_pad, b3_pad)
  return out


# speedup vs baseline: 6.4290x; 6.4290x over previous
import jax
import jax.numpy as jnp
from jax import lax
from jax.experimental import pallas as pl
from jax.experimental.pallas import tpu as pltpu
from jax.experimental.pallas import tpu_sc as plsc

N = 10000
F = 128
FH = 64
H2 = 256

NC = 2
NS = 16
NW = NC * NS

GW = 128
G_ALL = 2560
E_PAD = G_ALL * GW
GT = G_ALL // NS
GE = G_ALL // NW
N_PAD = 10240
RT = N_PAD // NS

_MESH = plsc.VectorSubcoreMesh(core_axis_name="c", subcore_axis_name="s",
                               num_cores=NC, num_subcores=NS)


def _agg_pipeline(x_view, src_v, dst_v, rows, sems, acc_sh, ngroups):
  pltpu.async_copy(x_view.at[src_v.at[0]], rows[0], sems[0])

  @pl.loop(0, ngroups, step=2)
  def _(g0):
    for b in range(2):
      g = g0 + b

      @pl.when(g + 1 < ngroups)
      def _():
        pltpu.async_copy(x_view.at[src_v.at[g + 1]], rows[1 - b],
                         sems[1 - b])

      pltpu.make_async_copy(x_view.at[src_v.at[g]], rows[b], sems[b]).wait()
      pltpu.sync_copy(rows[b], acc_sh.at[dst_v.at[g]], add=True)


def _colsplit_body(x_hbm, src_hbm, dst_hbm, zacc_hbm, out_hbm,
                   src_v, dst_v, rows0, rows1, acc_sh, sem0, sem1):
  c = lax.axis_index("c")
  s = lax.axis_index("s")

  pltpu.sync_copy(zacc_hbm, acc_sh.at[pl.ds(s * RT, RT)])
  pltpu.sync_copy(src_hbm.at[c].at[pl.ds(s * GT, GT)], src_v)
  pltpu.sync_copy(dst_hbm.at[pl.ds(s * GT, GT)], dst_v)
  plsc.subcore_barrier()

  _agg_pipeline(x_hbm, src_v, dst_v, (rows0, rows1), (sem0, sem1),
                acc_sh, GT)

  plsc.subcore_barrier()
  pltpu.sync_copy(acc_sh.at[pl.ds(s * RT, RT)],
                  out_hbm.at[c].at[pl.ds(s * RT, RT)])


_sc_agg_col = pl.kernel(
    _colsplit_body,
    out_type=jax.ShapeDtypeStruct((NC, N_PAD, FH), jnp.float32),
    mesh=_MESH,
    compiler_params=pltpu.CompilerParams(use_tc_tiling_on_sc=False),
    scratch_types=[
        pltpu.VMEM((GT, GW), jnp.int32),
        pltpu.VMEM((GT, GW), jnp.int32),
        pltpu.VMEM((GW, FH), jnp.float32),
        pltpu.VMEM((GW, FH), jnp.float32),
        pltpu.VMEM_SHARED((N_PAD, FH), jnp.float32),
        pltpu.SemaphoreType.DMA,
        pltpu.SemaphoreType.DMA,
    ],
)


def _edgesplit16_body(x_hbm, src_hbm, dst_hbm, zacc_hbm, out_hbm,
                      src_v, dst_v, rows0, rows1, acc_sh, sem0, sem1):
  c = lax.axis_index("c")
  s = lax.axis_index("s")
  wid = c * NS + s

  pltpu.sync_copy(zacc_hbm, acc_sh.at[pl.ds(s * RT, RT)])
  pltpu.sync_copy(src_hbm.at[pl.ds(wid * GE, GE)], src_v)
  pltpu.sync_copy(dst_hbm.at[pl.ds(wid * GE, GE)], dst_v)
  plsc.subcore_barrier()

  _agg_pipeline(x_hbm, src_v, dst_v, (rows0, rows1), (sem0, sem1),
                acc_sh, GE)

  plsc.subcore_barrier()
  pltpu.sync_copy(acc_sh.at[pl.ds(s * RT, RT)],
                  out_hbm.at[c].at[pl.ds(s * RT, RT)])


_sc_agg_16 = pl.kernel(
    _edgesplit16_body,
    out_type=jax.ShapeDtypeStruct((NC, N_PAD, 16), jnp.float32),
    mesh=_MESH,
    compiler_params=pltpu.CompilerParams(use_tc_tiling_on_sc=False),
    scratch_types=[
        pltpu.VMEM((GE, GW), jnp.int32),
        pltpu.VMEM((GE, GW), jnp.int32),
        pltpu.VMEM((GW, 16), jnp.float32),
        pltpu.VMEM((GW, 16), jnp.float32),
        pltpu.VMEM_SHARED((N_PAD, 16), jnp.float32),
        pltpu.SemaphoreType.DMA,
        pltpu.SemaphoreType.DMA,
    ],
)


def _counts_body(dst_hbm, zacc_hbm, ones_hbm, out_hbm,
                 dst_v, ones_v, cnt_sh):
  c = lax.axis_index("c")
  s = lax.axis_index("s")
  wid = c * NS + s

  pltpu.sync_copy(zacc_hbm, cnt_sh.at[pl.ds(s * RT, RT)])
  pltpu.sync_copy(dst_hbm.at[pl.ds(wid * GE, GE)], dst_v)
  pltpu.sync_copy(ones_hbm, ones_v)
  plsc.subcore_barrier()

  @pl.loop(0, GE)
  def _(g):
    pltpu.sync_copy(ones_v, cnt_sh.at[dst_v.at[g]], add=True)

  plsc.subcore_barrier()
  pltpu.sync_copy(cnt_sh.at[pl.ds(s * RT, RT)],
                  out_hbm.at[c].at[pl.ds(s * RT, RT)])


_sc_counts = pl.kernel(
    _counts_body,
    out_type=jax.ShapeDtypeStruct((NC, N_PAD, 16), jnp.float32),
    mesh=_MESH,
    compiler_params=pltpu.CompilerParams(use_tc_tiling_on_sc=False),
    scratch_types=[
        pltpu.VMEM((GE, GW), jnp.int32),
        pltpu.VMEM((GW, 16), jnp.float32),
        pltpu.VMEM_SHARED((N_PAD, 16), jnp.float32),
    ],
)


BR = 400


def _mean(agg, cnt_ref):
  deg = (cnt_ref[0] + cnt_ref[1])[:, 0:1]
  return agg * (1.0 / jnp.maximum(deg, 1.0))


def _halves(ref):
  return jnp.concatenate([ref[0], ref[1]], axis=1)


def _tc_layer1(agg_ref, cnt_ref, x_ref, wl_ref, wr_ref, b_ref, o_ref):
  mean = _mean(_halves(agg_ref), cnt_ref)
  h = jnp.dot(mean, wl_ref[...], preferred_element_type=jnp.float32)
  h = h + b_ref[...] + jnp.dot(x_ref[...], wr_ref[...],
                               preferred_element_type=jnp.float32)
  o_ref[...] = jnp.maximum(h, 0.0)


def _tc_layer2(agg_ref, cnt_ref, h1_ref, wl_ref, wr_ref, b_ref, w3l_ref,
               h2_ref, z_ref):
  mean = _mean(_halves(agg_ref), cnt_ref)
  h = jnp.dot(mean, wl_ref[...], preferred_element_type=jnp.float32)
  h = h + b_ref[...] + jnp.dot(h1_ref[...], wr_ref[...],
                               preferred_element_type=jnp.float32)
  h2 = jnp.maximum(h, 0.0)
  h2_ref[...] = h2
  z_ref[...] = jnp.dot(h2, w3l_ref[...], preferred_element_type=jnp.float32)


def _tc_layer3(part_ref, cnt_ref, h2_ref, w3r_ref, b3_ref, o_ref):
  agg = part_ref[0] + part_ref[1]
  mean = _mean(agg, cnt_ref)
  r = jnp.dot(h2_ref[...], w3r_ref[...], preferred_element_type=jnp.float32)
  logits = (mean + r + b3_ref[...])[:, 0:2]
  m = jnp.max(logits, axis=1, keepdims=True)
  lse = m + jnp.log(jnp.sum(jnp.exp(logits - m), axis=1, keepdims=True))
  o_ref[...] = logits - lse


def _row_spec(d):
  return pl.BlockSpec((BR, d), lambda i: (i, 0))


def _part_spec(d):
  return pl.BlockSpec((NC, BR, d), lambda i: (0, i, 0))


def _full(shape):
  return pl.BlockSpec(shape, lambda i: tuple(0 for _ in shape))


_GRID = (N // BR,)


def kernel(x, edge_index, W1_l, W1_r, b1, W2_l, W2_r, b2, W3_l, W3_r, b3):
  src = edge_index[0]
  dst = edge_index[1]
  npad = E_PAD - src.shape[0]
  src_p = jnp.concatenate([src, jnp.zeros((npad,), jnp.int32)])
  dst_p = jnp.concatenate(
      [dst, N + (jnp.arange(npad, dtype=jnp.int32) % (N_PAD - N))])
  src2d = src_p.reshape(G_ALL, GW)
  dst2d = dst_p.reshape(G_ALL, GW)
  src_off = jnp.stack([src2d, src2d + N])

  zacc64 = jnp.zeros((RT, FH), jnp.float32)
  zacc16 = jnp.zeros((RT, 16), jnp.float32)
  ones = jnp.ones((GW, 16), jnp.float32)

  cnt = _sc_counts(dst2d, zacc16, ones)

  x_flat = jnp.concatenate([x[:, :FH], x[:, FH:]], axis=0)
  agg1 = _sc_agg_col(x_flat, src_off, dst2d, zacc64)

  b1r = b1.reshape(1, F)
  h1 = pl.pallas_call(
      _tc_layer1,
      grid=_GRID,
      in_specs=[_part_spec(FH), _part_spec(16), _row_spec(F),
                _full((F, F)), _full((F, F)), _full((1, F))],
      out_specs=_row_spec(F),
      out_shape=jax.ShapeDtypeStruct((N, F), jnp.float32),
  )(agg1, cnt, x, W1_l, W1_r, b1r)

  h1_flat = jnp.concatenate([h1[:, :FH], h1[:, FH:]], axis=0)
  agg2 = _sc_agg_col(h1_flat, src_off, dst2d, zacc64)

  b2r = b2.reshape(1, H2)
  w3l_pad = jnp.pad(W3_l, ((0, 0), (0, 14)))
  h2, z = pl.pallas_call(
      _tc_layer2,
      grid=_GRID,
      in_specs=[_part_spec(FH), _part_spec(16), _row_spec(F),
                _full((F, H2)), _full((F, H2)), _full((1, H2)),
                _full((H2, 16))],
      out_specs=[_row_spec(H2), _row_spec(16)],
      out_shape=[jax.ShapeDtypeStruct((N, H2), jnp.float32),
                 jax.ShapeDtypeStruct((N, 16), jnp.float32)],
  )(agg2, cnt, h1, W2_l, W2_r, b2r, w3l_pad)

  part3 = _sc_agg_16(z, src2d, dst2d, zacc16)

  w3r_pad = jnp.pad(W3_r, ((0, 0), (0, 14)))
  b3_pad = jnp.pad(b3, (0, 14)).reshape(1, 16)
  out = pl.pallas_call(
      _tc_layer3,
      grid=_GRID,
      in_specs=[_part_spec(16), _part_spec(16), _row_spec(H2),
                _full((H2, 16)), _full((1, 16))],
      out_specs=pl.BlockSpec((BR, 2), lambda i: (i, 0)),
      out_shape=jax.ShapeDtypeStruct((N, 2), jnp.float32),
  )(part3, cnt, h2, w3r_pad, b3_pad)
  return out
